# baseline (speedup 1.0000x reference)
import jax
import jax.numpy as jnp
from jax.experimental import pallas as pl
from jax.experimental.pallas import tpu as pltpu

_BN_EPS = 1e-5
_VMEM_LIMIT = 64 * 1024 * 1024


def _make_conv_stats_kernel(h, w, cin, cout, kh_taps, kw_taps):
    hw = h * w

    def _conv_stats_kernel(x_ref, w_ref, ct_ref, s_ref, q_ref):
        xmb = x_ref[0].astype(jnp.bfloat16)

        zc = jnp.zeros((cin, w), jnp.bfloat16)
        xct = jnp.concatenate([
            jnp.concatenate([zc, xmb[:, :-w]], axis=1),
            xmb,
            jnp.concatenate([xmb[:, w:], zc], axis=1),
        ], axis=0)

        xcp = xct.T

        pix = jax.lax.broadcasted_iota(jnp.int32, (hw, 1), 0)
        col = pix % w
        z1 = jnp.zeros((1, kh_taps * cin), jnp.bfloat16)
        xm = jnp.where(col == 0, jnp.bfloat16(0),
                       jnp.concatenate([z1, xcp[:-1]], axis=0))
        xp = jnp.where(col == w - 1, jnp.bfloat16(0),
                       jnp.concatenate([xcp[1:], z1], axis=0))

        acc = jnp.zeros((hw, cout), jnp.float32)
        for kw, patch in zip(range(kw_taps), (xm, xcp, xp)):
            acc += jax.lax.dot_general(
                patch, w_ref[kw],
                dimension_numbers=(((1,), (0,)), ((), ())),
                preferred_element_type=jnp.float32)

        s_ref[...] = jnp.sum(acc, axis=0).reshape(1, 1, cout)
        q_ref[...] = jnp.sum(acc * acc, axis=0).reshape(1, 1, cout)
        ct_ref[...] = acc.T.astype(jnp.bfloat16).reshape(1, cout, hw)

    return _conv_stats_kernel


def _bn_relu_kernel(ct_ref, sc_ref, sh_ref, o_ref):
    v = ct_ref[0].astype(jnp.float32)
    y = jnp.maximum(v * sc_ref[...] + sh_ref[...], 0.0)
    o_ref[...] = y.reshape(o_ref.shape)


def kernel(x_nchw, w_oihw, bias, gamma, beta):
    del bias
    n, cin, h, w = x_nchw.shape
    cout, _, kh, kw = w_oihw.shape
    hw = h * w

    xf = x_nchw.reshape(n, cin, hw)

    w3 = jnp.transpose(w_oihw, (3, 2, 1, 0)).reshape(kw, kh * cin, cout)
    w3 = w3.astype(jnp.bfloat16)

    conv_flops = 2 * n * hw * kh * kw * cin * cout
    conv_bytes = 4 * xf.size + 2 * (w3.size + n * hw * cout)

    convt, csum, csq = pl.pallas_call(
        _make_conv_stats_kernel(h, w, cin, cout, kh, kw),
        grid=(n,),
        in_specs=[
            pl.BlockSpec((1, cin, hw), lambda i: (i, 0, 0)),
            pl.BlockSpec((kw, kh * cin, cout), lambda i: (0, 0, 0)),
        ],
        out_specs=[
            pl.BlockSpec((1, cout, hw), lambda i: (i, 0, 0)),
            pl.BlockSpec((1, 1, cout), lambda i: (i, 0, 0)),
            pl.BlockSpec((1, 1, cout), lambda i: (i, 0, 0)),
        ],
        out_shape=(
            jax.ShapeDtypeStruct((n, cout, hw), jnp.bfloat16),
            jax.ShapeDtypeStruct((n, 1, cout), jnp.float32),
            jax.ShapeDtypeStruct((n, 1, cout), jnp.float32),
        ),
        compiler_params=pltpu.CompilerParams(
            dimension_semantics=("parallel",),
            vmem_limit_bytes=_VMEM_LIMIT),
        cost_estimate=pl.CostEstimate(
            flops=conv_flops, transcendentals=0, bytes_accessed=conv_bytes),
    )(xf, w3)

    cnt = float(n * hw)
    mean = csum.sum(axis=(0, 1)) / cnt
    var = jnp.maximum(csq.sum(axis=(0, 1)) / cnt - mean * mean, 0.0)
    scale = gamma.astype(jnp.float32) * jax.lax.rsqrt(var + _BN_EPS)
    shift = beta.astype(jnp.float32) - mean * scale

    out = pl.pallas_call(
        _bn_relu_kernel,
        grid=(n,),
        in_specs=[
            pl.BlockSpec((1, cout, hw), lambda i: (i, 0, 0)),
            pl.BlockSpec((cout, 1), lambda i: (0, 0)),
            pl.BlockSpec((cout, 1), lambda i: (0, 0)),
        ],
        out_specs=pl.BlockSpec((1, cout, hw), lambda i: (i, 0, 0)),
        out_shape=jax.ShapeDtypeStruct((n, cout, hw), jnp.float32),
        compiler_params=pltpu.CompilerParams(
            dimension_semantics=("parallel",),
            vmem_limit_bytes=_VMEM_LIMIT),
        cost_estimate=pl.CostEstimate(
            flops=3 * n * hw * cout, transcendentals=0,
            bytes_accessed=6 * n * hw * cout),
    )(convt, scale.reshape(cout, 1), shift.reshape(cout, 1))

    return out.reshape(n, cout, h, w)
